# Initial kernel scaffold; baseline (speedup 1.0000x reference)
import jax
import jax.numpy as jnp
from jax.experimental import pallas as pl


def kernel(net_output, target):
    raise NotImplementedError("write your pallas kernel here")


# speedup vs baseline: 2.7101x; 2.7101x over previous
import jax
import jax.numpy as jnp
from jax.experimental import pallas as pl
from jax.experimental.pallas import tpu as pltpu

_SMOOTH = 1e-05

_ROWS = 16384
_LANES = 128
_BLK = 2048
_STEPS = _ROWS // _BLK


def _dice_kernel(net_ref, tgt_ref, out_ref, acc_ref):
    b = pl.program_id(0)
    j = pl.program_id(1)

    @pl.when(jnp.logical_and(b == 0, j == 0))
    def _init():
        for q in range(3):
            for bb in range(2):
                for ci in range(3):
                    acc_ref[q, bb, ci] = jnp.float32(0.0)

    x0 = net_ref[0, 0]
    x1 = net_ref[0, 1]
    x2 = net_ref[0, 2]
    x3 = net_ref[0, 3]
    m = jnp.maximum(jnp.maximum(x0, x1), jnp.maximum(x2, x3))
    e0 = jnp.exp(x0 - m)
    e1 = jnp.exp(x1 - m)
    e2 = jnp.exp(x2 - m)
    e3 = jnp.exp(x3 - m)
    inv = 1.0 / (e0 + e1 + e2 + e3)
    t = tgt_ref[0]

    for ci, e in ((0, e1), (1, e2), (2, e3)):
        p = e * inv
        mask = t == (ci + 1)
        tp = jnp.sum(jnp.where(mask, p, 0.0))
        sp = jnp.sum(p)
        cnt = jnp.sum(jnp.where(mask, 1.0, 0.0))
        acc_ref[0, b, ci] += tp
        acc_ref[1, b, ci] += sp
        acc_ref[2, b, ci] += cnt

    @pl.when(jnp.logical_and(b == pl.num_programs(0) - 1,
                             j == pl.num_programs(1) - 1))
    def _finish():
        loss = jnp.float32(0.0)
        for bb in range(2):
            for ci in range(3):
                tp = acc_ref[0, bb, ci]
                sp = acc_ref[1, bb, ci]
                cnt = acc_ref[2, bb, ci]
                dice = (2.0 * tp + _SMOOTH) / (sp + cnt + _SMOOTH)
                loss += 1.0 - dice
        out_ref[0, 0] = loss / 6.0


def kernel(net_output, target):
    n = net_output.reshape(2, 4, _ROWS, _LANES)
    t = target.reshape(2, _ROWS, _LANES)
    out = pl.pallas_call(
        _dice_kernel,
        grid=(2, _STEPS),
        in_specs=[
            pl.BlockSpec((1, 4, _BLK, _LANES), lambda b, j: (b, 0, j, 0)),
            pl.BlockSpec((1, _BLK, _LANES), lambda b, j: (b, j, 0)),
        ],
        out_specs=pl.BlockSpec(memory_space=pltpu.SMEM),
        out_shape=jax.ShapeDtypeStruct((1, 1), jnp.float32),
        scratch_shapes=[pltpu.SMEM((3, 2, 3), jnp.float32)],
    )(n, t)
    return out[0, 0]
